# Initial kernel scaffold; baseline (speedup 1.0000x reference)
import jax
import jax.numpy as jnp
from jax.experimental import pallas as pl


def kernel(x, edge_index, edge_attr, W1, root1, b1, W2, root2, b2, W3, root3, b3, fc_w, fc_b):
    raise NotImplementedError("write your pallas kernel here")


# speedup vs baseline: 2.7127x; 2.7127x over previous
import functools

import jax
import jax.numpy as jnp
from jax import lax
from jax.experimental import pallas as pl
from jax.experimental.pallas import tpu as pltpu
from jax.experimental.pallas import tpu_sc as plsc

N = 10000
E = 160000
K = 25
F = 16

NC, NS = 2, 16
NW = NC * NS
R = E * 4
GR = R // 128
CPW = GR // NW
GREM = GR - CPW * NW
ER = E // 128
RPC = ER // NC
RPT = RPC // NS
NPT = 624
NPT_LAST = N - (NS - 1) * NPT

_mesh = plsc.VectorSubcoreMesh(
    core_axis_name="c", subcore_axis_name="s", num_cores=NC, num_subcores=NS)


def _basis_body(ea_ref, src_ref, b_ref, gi_ref):
    v = ea_ref[...] * 4.0
    lo = jnp.floor(v)
    fr = v - lo
    li = lo.astype(jnp.int32)
    i0, i1 = li[:, 0:1], li[:, 1:2]
    f0, f1 = fr[:, 0:1], fr[:, 1:2]
    i0b = jnp.minimum(i0 + 1, 4)
    i1b = jnp.minimum(i1 + 1, 4)
    b_ref[...] = jnp.concatenate(
        [(1 - f0) * (1 - f1), (1 - f0) * f1, f0 * (1 - f1), f0 * f1], axis=1)
    s25 = src_ref[...] * K
    gi_ref[...] = jnp.concatenate(
        [s25 + i0 + 5 * i1, s25 + i0 + 5 * i1b,
         s25 + i0b + 5 * i1, s25 + i0b + 5 * i1b], axis=1)


_EB = 8000
_basis_call = pl.pallas_call(
    _basis_body,
    grid=(E // _EB,),
    in_specs=[pl.BlockSpec((_EB, 2), lambda i: (i, 0)),
              pl.BlockSpec((_EB, 1), lambda i: (i, 0))],
    out_specs=[pl.BlockSpec((_EB, 4), lambda i: (i, 0)),
               pl.BlockSpec((_EB, 4), lambda i: (i, 0))],
    out_shape=[jax.ShapeDtypeStruct((E, 4), jnp.float32),
               jax.ShapeDtypeStruct((E, 4), jnp.int32)],
)


def _mm_body(h_ref, w_ref, o_ref):
    o_ref[...] = jnp.dot(h_ref[...], w_ref[...],
                         preferred_element_type=jnp.float32)


_NB = 2000
_mm_call = pl.pallas_call(
    _mm_body,
    grid=(N // _NB,),
    in_specs=[pl.BlockSpec((_NB, F), lambda i: (i, 0)),
              pl.BlockSpec((F, K * F), lambda i: (0, 0))],
    out_specs=pl.BlockSpec((_NB, K * F), lambda i: (i, 0)),
    out_shape=jax.ShapeDtypeStruct((N, K * F), jnp.float32),
)


def _gather_body(gidx, table, out, idxv, rowsv, gsem, wsem):
    w = lax.axis_index("s") * NC + lax.axis_index("c")
    base_i = pl.multiple_of(w * (CPW * 128), 128)
    pltpu.sync_copy(gidx.at[pl.ds(base_i, CPW * 128)],
                    idxv.at[pl.ds(0, CPW * 128)])

    def grp(g, carry):
        b0 = pl.multiple_of(g * 512, 512)
        cps = [pltpu.async_copy(
                   table.at[idxv.at[pl.ds(b0 + i * 128, 128)]],
                   rowsv.at[pl.ds(i * 128, 128)], gsem)
               for i in range(4)]
        for cp in cps:
            cp.wait()
        pltpu.async_copy(
            rowsv, out.at[pl.ds(pl.multiple_of(base_i + b0, 128), 512)],
            wsem).wait()
        return carry

    lax.fori_loop(0, CPW // 4, grp, 0)

    @pl.when(w < GREM)
    def _():
        p0 = pl.multiple_of(NW * CPW * 128 + w * 128, 128)
        pltpu.sync_copy(gidx.at[pl.ds(p0, 128)],
                        idxv.at[pl.ds(CPW * 128, 128)])
        pltpu.async_copy(table.at[idxv.at[pl.ds(CPW * 128, 128)]],
                         rowsv.at[pl.ds(0, 128)], gsem).wait()
        pltpu.async_copy(rowsv.at[pl.ds(0, 128)],
                         out.at[pl.ds(p0, 128)], wsem).wait()


_gather_call = functools.partial(
    pl.kernel,
    out_type=jax.ShapeDtypeStruct((R, F), jnp.float32),
    mesh=_mesh,
    compiler_params=pltpu.CompilerParams(use_tc_tiling_on_sc=False),
    scratch_types=[pltpu.VMEM(((CPW + 1) * 128,), jnp.int32),
                   pltpu.VMEM((512, F), jnp.float32),
                   pltpu.SemaphoreType.DMA,
                   pltpu.SemaphoreType.DMA],
)(_gather_body)


def _wsum_body(g_ref, b_ref, o_ref):
    b = b_ref[...]
    acc = b[:, 0:1] * g_ref[:, 0, :]
    for j in range(1, 4):
        acc = acc + b[:, j:j + 1] * g_ref[:, j, :]
    o_ref[...] = acc


_WB = 2000
_wsum_call = pl.pallas_call(
    _wsum_body,
    grid=(E // _WB,),
    in_specs=[pl.BlockSpec((_WB, 4, F), lambda i: (i, 0, 0)),
              pl.BlockSpec((_WB, 4), lambda i: (i, 0))],
    out_specs=pl.BlockSpec((_WB, F), lambda i: (i, 0)),
    out_shape=jax.ShapeDtypeStruct((E, F), jnp.float32),
)


def _init_acc(zeros, acc, s):
    nb = pl.multiple_of(s * NPT, 8)

    @pl.when(s < NS - 1)
    def _():
        pltpu.sync_copy(zeros.at[pl.ds(nb, NPT)], acc.at[pl.ds(nb, NPT)])

    @pl.when(s == NS - 1)
    def _():
        b0 = (NS - 1) * NPT
        pltpu.sync_copy(zeros.at[pl.ds(b0, NPT_LAST)],
                        acc.at[pl.ds(b0, NPT_LAST)])


def _copy_out(acc, out, c, s):
    nb = pl.multiple_of(s * NPT, 8)

    @pl.when(s < NS - 1)
    def _():
        pltpu.sync_copy(acc.at[pl.ds(nb, NPT)],
                        out.at[pl.ds(pl.multiple_of(c * N + nb, 8), NPT)])

    @pl.when(s == NS - 1)
    def _():
        b0 = (NS - 1) * NPT
        pltpu.sync_copy(
            acc.at[pl.ds(b0, NPT_LAST)],
            out.at[pl.ds(pl.multiple_of(c * N + b0, 8), NPT_LAST)])


def _scatter_body(dst1, msg, zeros, out, didx, mbuf, acc, msem):
    c = lax.axis_index("c")
    s = lax.axis_index("s")
    _init_acc(zeros, acc, s)
    rbase = c * RPC + s * RPT
    plsc.subcore_barrier()

    def chunk(b, carry):
        off = pl.multiple_of((rbase + b) * 128, 128)
        pltpu.sync_copy(dst1.at[pl.ds(off, 128)], didx.at[0])
        pltpu.sync_copy(msg.at[pl.ds(off, 128)], mbuf)
        pltpu.sync_copy(mbuf, acc.at[didx.at[0]], add=True)
        return carry

    lax.fori_loop(0, RPT, chunk, 0)

    @pl.when(s == NS - 1)
    def _():
        off = pl.multiple_of((c * RPC + NS * RPT) * 128, 128)
        pltpu.sync_copy(dst1.at[pl.ds(off, 128)], didx.at[0])
        pltpu.sync_copy(msg.at[pl.ds(off, 128)], mbuf)
        pltpu.sync_copy(mbuf, acc.at[didx.at[0]], add=True)

    plsc.subcore_barrier()
    _copy_out(acc, out, c, s)


_scatter_call = functools.partial(
    pl.kernel,
    out_type=jax.ShapeDtypeStruct((NC * N, F), jnp.float32),
    mesh=_mesh,
    compiler_params=pltpu.CompilerParams(use_tc_tiling_on_sc=False),
    scratch_types=[pltpu.VMEM((1, 128), jnp.int32),
                   pltpu.VMEM((128, F), jnp.float32),
                   pltpu.VMEM_SHARED((N, F), jnp.float32),
                   pltpu.SemaphoreType.DMA],
)(_scatter_body)


def _deg_body(dst1, ones128, zeros, out, didx, obuf, acc, msem):
    c = lax.axis_index("c")
    s = lax.axis_index("s")
    _init_acc(zeros, acc, s)
    pltpu.sync_copy(ones128, obuf)
    rbase = c * RPC + s * RPT
    plsc.subcore_barrier()

    def chunk(b, carry):
        off = pl.multiple_of((rbase + b) * 128, 128)
        pltpu.sync_copy(dst1.at[pl.ds(off, 128)], didx.at[0])
        pltpu.sync_copy(obuf, acc.at[didx.at[0]], add=True)
        return carry

    lax.fori_loop(0, RPT, chunk, 0)

    @pl.when(s == NS - 1)
    def _():
        off = pl.multiple_of((c * RPC + NS * RPT) * 128, 128)
        pltpu.sync_copy(dst1.at[pl.ds(off, 128)], didx.at[0])
        pltpu.sync_copy(obuf, acc.at[didx.at[0]], add=True)

    plsc.subcore_barrier()
    _copy_out(acc, out, c, s)


_deg_call = functools.partial(
    pl.kernel,
    out_type=jax.ShapeDtypeStruct((NC * N, F), jnp.float32),
    mesh=_mesh,
    compiler_params=pltpu.CompilerParams(use_tc_tiling_on_sc=False),
    scratch_types=[pltpu.VMEM((1, 128), jnp.int32),
                   pltpu.VMEM((128, F), jnp.float32),
                   pltpu.VMEM_SHARED((N, F), jnp.float32),
                   pltpu.SemaphoreType.DMA],
)(_deg_body)


def _layerend_body(agg_ref, deg_ref, h_ref, root_ref, bias_ref, o_ref):
    a = agg_ref[0] + agg_ref[1]
    deg = deg_ref[0, :, 0:1] + deg_ref[1, :, 0:1]
    a = a / jnp.maximum(deg, 1.0)
    o_ref[...] = jax.nn.relu(
        a + jnp.dot(h_ref[...], root_ref[...],
                    preferred_element_type=jnp.float32) + bias_ref[...])


_layerend_call = pl.pallas_call(
    _layerend_body,
    grid=(N // _NB,),
    in_specs=[pl.BlockSpec((NC, _NB, F), lambda i: (0, i, 0)),
              pl.BlockSpec((NC, _NB, F), lambda i: (0, i, 0)),
              pl.BlockSpec((_NB, F), lambda i: (i, 0)),
              pl.BlockSpec((F, F), lambda i: (0, 0)),
              pl.BlockSpec((1, F), lambda i: (0, 0))],
    out_specs=pl.BlockSpec((_NB, F), lambda i: (i, 0)),
    out_shape=jax.ShapeDtypeStruct((N, F), jnp.float32),
)


def _final_body(h_ref, fw_ref, fb_ref, o_ref):
    o_ref[...] = jax.nn.sigmoid(
        jnp.dot(h_ref[...], fw_ref[...],
                preferred_element_type=jnp.float32) + fb_ref[...])


_final_call = pl.pallas_call(
    _final_body,
    grid=(N // _NB,),
    in_specs=[pl.BlockSpec((_NB, F), lambda i: (i, 0)),
              pl.BlockSpec((F, 1), lambda i: (0, 0)),
              pl.BlockSpec((1, 1), lambda i: (0, 0))],
    out_specs=pl.BlockSpec((_NB, 1), lambda i: (i, 0)),
    out_shape=jax.ShapeDtypeStruct((N, 1), jnp.float32),
)


def kernel(x, edge_index, edge_attr, W1, root1, b1, W2, root2, b2,
           W3, root3, b3, fc_w, fc_b):
    f32 = jnp.float32
    src = edge_index[0].reshape(E, 1)
    dst1 = edge_index[1]

    B2d, gidx = _basis_call(edge_attr, src)
    gidx1 = gidx.reshape(R)

    zeros = jnp.zeros((N, F), f32)
    ones128 = jnp.ones((128, F), f32)
    deg3 = _deg_call(dst1, ones128, zeros).reshape(NC, N, F)

    xp = jnp.pad(x, ((0, 0), (0, F - 2)))
    Wr1 = jnp.pad(jnp.transpose(W1, (1, 0, 2)).reshape(2, K * F),
                  ((0, F - 2), (0, 0)))
    r1p = jnp.pad(root1, ((0, F - 2), (0, 0)))
    Wr2 = jnp.transpose(W2, (1, 0, 2)).reshape(F, K * F)
    Wr3 = jnp.transpose(W3, (1, 0, 2)).reshape(F, K * F)

    h = xp
    for Wr, root, bias in ((Wr1, r1p, b1), (Wr2, root2, b2), (Wr3, root3, b3)):
        table = _mm_call(h, Wr).reshape(N * K, F)
        G = _gather_call(gidx1, table)
        msg = _wsum_call(G.reshape(E, 4, F), B2d)
        aggf = _scatter_call(dst1, msg, zeros)
        h = _layerend_call(aggf.reshape(NC, N, F), deg3, h, root,
                           bias.reshape(1, F))

    out = _final_call(h, fc_w, fc_b.reshape(1, 1))
    return out[:, 0]


# speedup vs baseline: 9.3305x; 3.4395x over previous
import functools

import jax
import jax.numpy as jnp
from jax import lax
from jax.experimental import pallas as pl
from jax.experimental.pallas import tpu as pltpu
from jax.experimental.pallas import tpu_sc as plsc

N = 10000
E = 160000
K = 25
F = 16

NC, NS = 2, 16
NW = NC * NS
R = E * 4
GR = R // 128
CPW = GR // NW
GREM = GR - CPW * NW
ER = E // 128
RPC = ER // NC
RPT = RPC // NS
NPT = 624
NPT_LAST = N - (NS - 1) * NPT

_mesh = plsc.VectorSubcoreMesh(
    core_axis_name="c", subcore_axis_name="s", num_cores=NC, num_subcores=NS)


def _basis_body(eat_ref, ei_ref, g_ref, b_ref, d_ref):
    ea0 = eat_ref[0, :]
    ea1 = eat_ref[1, :]
    src = ei_ref[0, :]
    v0 = ea0 * 4.0
    v1 = ea1 * 4.0
    lo0 = jnp.floor(v0)
    lo1 = jnp.floor(v1)
    f0 = v0 - lo0
    f1 = v1 - lo1
    i0 = lo0.astype(jnp.int32)
    i1 = lo1.astype(jnp.int32)
    i0b = jnp.minimum(i0 + 1, 4)
    i1b = jnp.minimum(i1 + 1, 4)
    s25 = src * K
    b_ref[...] = jnp.concatenate(
        [((1 - f0) * (1 - f1))[None], ((1 - f0) * f1)[None],
         (f0 * (1 - f1))[None], (f0 * f1)[None]], axis=0)
    g_ref[...] = jnp.concatenate(
        [(s25 + i0 + 5 * i1)[None], (s25 + i0 + 5 * i1b)[None],
         (s25 + i0b + 5 * i1)[None], (s25 + i0b + 5 * i1b)[None]], axis=0)
    d_ref[...] = ei_ref[1:2, :]


_EB = 16000
_NEB = E // _EB
_basis_call = pl.pallas_call(
    _basis_body,
    grid=(_NEB,),
    in_specs=[pl.BlockSpec((2, _EB), lambda i: (0, i)),
              pl.BlockSpec((2, _EB), lambda i: (0, i))],
    out_specs=[pl.BlockSpec((4, _EB), lambda i: (0, i)),
               pl.BlockSpec((4, _EB), lambda i: (0, i)),
               pl.BlockSpec((1, _EB), lambda i: (0, i))],
    out_shape=[jax.ShapeDtypeStruct((4, E), jnp.int32),
               jax.ShapeDtypeStruct((4, E), jnp.float32),
               jax.ShapeDtypeStruct((1, E), jnp.int32)],
)


def _mm_body(h_ref, w_ref, o_ref):
    o_ref[...] = jnp.dot(h_ref[...], w_ref[...],
                         preferred_element_type=jnp.float32)


_NB = 2000
_mm_call = pl.pallas_call(
    _mm_body,
    grid=(N // _NB,),
    in_specs=[pl.BlockSpec((_NB, F), lambda i: (i, 0)),
              pl.BlockSpec((F, K * F), lambda i: (0, 0))],
    out_specs=pl.BlockSpec((_NB, K * F), lambda i: (i, 0)),
    out_shape=jax.ShapeDtypeStruct((N, K * F), jnp.float32),
)


SEG = (RPT + 1) * 128


def _weight_chunk(gbuf, bvals, mbuf, p, b):
    iota = lax.iota(jnp.int32, 16)
    for t in range(8):
        srow = p * 128 + t * 16 + iota
        rows = [p * 512 + j * 128 + t * 16 + iota for j in range(4)]
        bv = [bvals[pl.ds(j * SEG + b * 128 + t * 16, 16)] for j in range(4)]
        for i in range(F):
            col = jnp.full((16,), i, jnp.int32)
            g0 = plsc.load_gather(gbuf, [rows[0], col])
            g1 = plsc.load_gather(gbuf, [rows[1], col])
            g2 = plsc.load_gather(gbuf, [rows[2], col])
            g3 = plsc.load_gather(gbuf, [rows[3], col])
            acc = (bv[0] * g0 + bv[1] * g1) + (bv[2] * g2 + bv[3] * g3)
            plsc.store_scatter(mbuf, [srow, col], acc)


def _layer_body(gflat, bflat, dstc, table, zeros, out,
                idxv, bvals, didx, gbuf, mbuf, acc,
                gsem0, gsem1, ssem0, ssem1):
    c = lax.axis_index("c")
    s = lax.axis_index("s")
    rbase = c * RPC + s * RPT
    ebase = pl.multiple_of(rbase * 128, 128)
    nb = RPT + jnp.where(s == NS - 1, 1, 0)

    for j in range(4):
        pltpu.sync_copy(gflat.at[j, pl.ds(ebase, RPT * 128)],
                        idxv.at[pl.ds(j * SEG, RPT * 128)])
        pltpu.sync_copy(bflat.at[j, pl.ds(ebase, RPT * 128)],
                        bvals.at[pl.ds(j * SEG, RPT * 128)])

    @pl.when(s == NS - 1)
    def _():
        e2 = pl.multiple_of((c * RPC + NS * RPT) * 128, 128)
        for j in range(4):
            pltpu.sync_copy(gflat.at[j, pl.ds(e2, 128)],
                            idxv.at[pl.ds(j * SEG + RPT * 128, 128)])
            pltpu.sync_copy(bflat.at[j, pl.ds(e2, 128)],
                            bvals.at[pl.ds(j * SEG + RPT * 128, 128)])

    _init_acc(zeros, acc, s)
    plsc.subcore_barrier()

    def _fire(b, p, sem):
        for j in range(4):
            pltpu.async_copy(
                table.at[idxv.at[pl.ds(pl.multiple_of(j * SEG + b * 128, 128),
                                       128)]],
                gbuf.at[pl.ds(p * 512 + j * 128, 128)], sem)
        pltpu.async_copy(
            dstc.at[0, pl.ds(pl.multiple_of((rbase + b) * 128, 128), 128)],
            didx.at[lax.rem(b, 4)], sem)

    def _drain(b, p, sem):
        for j in range(4):
            pltpu.make_async_copy(
                table.at[idxv.at[pl.ds(pl.multiple_of(j * SEG + b * 128, 128),
                                       128)]],
                gbuf.at[pl.ds(p * 512 + j * 128, 128)], sem).wait()
        pltpu.make_async_copy(
            dstc.at[0, pl.ds(pl.multiple_of((rbase + b) * 128, 128), 128)],
            didx.at[lax.rem(b, 4)], sem).wait()

    _fire(0, 0, gsem0)

    def step(b, carry):
        p = lax.rem(b, 2)

        def _body(p_lit, gsem, ssem):
            @pl.when(b >= 2)
            def _():
                pltpu.make_async_copy(
                    mbuf.at[pl.ds(p_lit * 128, 128)],
                    acc.at[didx.at[lax.rem(b - 2, 4)]], ssem).wait()

            @pl.when(b + 1 < nb)
            def _():
                _fire(b + 1, 1 - p_lit, gsem1 if p_lit == 0 else gsem0)

            _drain(b, p_lit, gsem)
            _weight_chunk(gbuf, bvals, mbuf, p_lit, b)
            pltpu.async_copy(mbuf.at[pl.ds(p_lit * 128, 128)],
                             acc.at[didx.at[lax.rem(b, 4)]], ssem,
                             add=True)

        @pl.when(p == 0)
        def _():
            _body(0, gsem0, ssem0)

        @pl.when(p == 1)
        def _():
            _body(1, gsem1, ssem1)

        return carry

    lax.fori_loop(0, nb, step, 0)
    pltpu.make_async_copy(mbuf.at[pl.ds(0, 128)],
                          acc.at[didx.at[0]], ssem0).wait()
    pltpu.make_async_copy(mbuf.at[pl.ds(128, 128)],
                          acc.at[didx.at[1]], ssem1).wait()
    plsc.subcore_barrier()
    _copy_out(acc, out, c, s)


_layer_call = functools.partial(
    pl.kernel,
    out_type=jax.ShapeDtypeStruct((NC * N, F), jnp.float32),
    mesh=_mesh,
    compiler_params=pltpu.CompilerParams(use_tc_tiling_on_sc=False, needs_layout_passes=False),
    scratch_types=[pltpu.VMEM((4 * SEG,), jnp.int32),
                   pltpu.VMEM((4 * SEG,), jnp.float32),
                   pltpu.VMEM((4, 128), jnp.int32),
                   pltpu.VMEM((1024, F), jnp.float32),
                   pltpu.VMEM((256, F), jnp.float32),
                   pltpu.VMEM_SHARED((N, F), jnp.float32),
                   pltpu.SemaphoreType.DMA,
                   pltpu.SemaphoreType.DMA,
                   pltpu.SemaphoreType.DMA,
                   pltpu.SemaphoreType.DMA],
)(_layer_body)


def _init_acc(zeros, acc, s):
    nb = pl.multiple_of(s * NPT, 8)

    @pl.when(s < NS - 1)
    def _():
        pltpu.sync_copy(zeros.at[pl.ds(nb, NPT)], acc.at[pl.ds(nb, NPT)])

    @pl.when(s == NS - 1)
    def _():
        b0 = (NS - 1) * NPT
        pltpu.sync_copy(zeros.at[pl.ds(b0, NPT_LAST)],
                        acc.at[pl.ds(b0, NPT_LAST)])


def _copy_out(acc, out, c, s):
    nb = pl.multiple_of(s * NPT, 8)

    @pl.when(s < NS - 1)
    def _():
        pltpu.sync_copy(acc.at[pl.ds(nb, NPT)],
                        out.at[pl.ds(pl.multiple_of(c * N + nb, 8), NPT)])

    @pl.when(s == NS - 1)
    def _():
        b0 = (NS - 1) * NPT
        pltpu.sync_copy(
            acc.at[pl.ds(b0, NPT_LAST)],
            out.at[pl.ds(pl.multiple_of(c * N + b0, 8), NPT_LAST)])


def _deg_body(dst1, ones128, zeros, out, didx, obuf, acc, msem):
    c = lax.axis_index("c")
    s = lax.axis_index("s")
    _init_acc(zeros, acc, s)
    pltpu.sync_copy(ones128, obuf)
    rbase = c * RPC + s * RPT
    plsc.subcore_barrier()

    def chunk(b, carry):
        off = pl.multiple_of((rbase + b) * 128, 128)
        pltpu.sync_copy(dst1.at[0, pl.ds(off, 128)], didx.at[0])
        pltpu.sync_copy(obuf, acc.at[didx.at[0]], add=True)
        return carry

    lax.fori_loop(0, RPT, chunk, 0)

    @pl.when(s == NS - 1)
    def _():
        off = pl.multiple_of((c * RPC + NS * RPT) * 128, 128)
        pltpu.sync_copy(dst1.at[0, pl.ds(off, 128)], didx.at[0])
        pltpu.sync_copy(obuf, acc.at[didx.at[0]], add=True)

    plsc.subcore_barrier()
    _copy_out(acc, out, c, s)


_deg_call = functools.partial(
    pl.kernel,
    out_type=jax.ShapeDtypeStruct((NC * N, F), jnp.float32),
    mesh=_mesh,
    compiler_params=pltpu.CompilerParams(use_tc_tiling_on_sc=False, needs_layout_passes=False),
    scratch_types=[pltpu.VMEM((1, 128), jnp.int32),
                   pltpu.VMEM((128, F), jnp.float32),
                   pltpu.VMEM_SHARED((N, F), jnp.float32),
                   pltpu.SemaphoreType.DMA],
)(_deg_body)


def _relu_layer(agg_ref, deg_ref, h_ref, root_ref, bias_ref):
    a = agg_ref[0] + agg_ref[1]
    deg = deg_ref[0, :, 0:1] + deg_ref[1, :, 0:1]
    a = a / jnp.maximum(deg, 1.0)
    return jax.nn.relu(
        a + jnp.dot(h_ref[...], root_ref[...],
                    preferred_element_type=jnp.float32) + bias_ref[...])


def _lem_body(agg_ref, deg_ref, h_ref, root_ref, bias_ref, wr_ref,
              hn_ref, tab_ref):
    hn = _relu_layer(agg_ref, deg_ref, h_ref, root_ref, bias_ref)
    hn_ref[...] = hn
    tab_ref[...] = jnp.dot(hn, wr_ref[...], preferred_element_type=jnp.float32)


_lem_call = pl.pallas_call(
    _lem_body,
    grid=(N // _NB,),
    in_specs=[pl.BlockSpec((NC, _NB, F), lambda i: (0, i, 0)),
              pl.BlockSpec((NC, _NB, F), lambda i: (0, i, 0)),
              pl.BlockSpec((_NB, F), lambda i: (i, 0)),
              pl.BlockSpec((F, F), lambda i: (0, 0)),
              pl.BlockSpec((1, F), lambda i: (0, 0)),
              pl.BlockSpec((F, K * F), lambda i: (0, 0))],
    out_specs=[pl.BlockSpec((_NB, F), lambda i: (i, 0)),
               pl.BlockSpec((_NB, K * F), lambda i: (i, 0))],
    out_shape=[jax.ShapeDtypeStruct((N, F), jnp.float32),
               jax.ShapeDtypeStruct((N, K * F), jnp.float32)],
)


def _lef_body(agg_ref, deg_ref, h_ref, root_ref, bias_ref, fw_ref, fb_ref,
              o_ref):
    hn = _relu_layer(agg_ref, deg_ref, h_ref, root_ref, bias_ref)
    o_ref[...] = jax.nn.sigmoid(
        jnp.dot(hn, fw_ref[...],
                preferred_element_type=jnp.float32) + fb_ref[...])


_lef_call = pl.pallas_call(
    _lef_body,
    grid=(N // _NB,),
    in_specs=[pl.BlockSpec((NC, _NB, F), lambda i: (0, i, 0)),
              pl.BlockSpec((NC, _NB, F), lambda i: (0, i, 0)),
              pl.BlockSpec((_NB, F), lambda i: (i, 0)),
              pl.BlockSpec((F, F), lambda i: (0, 0)),
              pl.BlockSpec((1, F), lambda i: (0, 0)),
              pl.BlockSpec((F, 1), lambda i: (0, 0)),
              pl.BlockSpec((1, 1), lambda i: (0, 0))],
    out_specs=pl.BlockSpec((_NB, 1), lambda i: (i, 0)),
    out_shape=jax.ShapeDtypeStruct((N, 1), jnp.float32),
)


def kernel(x, edge_index, edge_attr, W1, root1, b1, W2, root2, b2,
           W3, root3, b3, fc_w, fc_b):
    f32 = jnp.float32
    eat = edge_attr.T

    gflat, bflat, dstc = _basis_call(eat, edge_index)

    zeros = jnp.zeros((N, F), f32)
    ones128 = jnp.ones((128, F), f32)
    deg3 = _deg_call(dstc, ones128, zeros).reshape(NC, N, F)

    xp = jnp.pad(x, ((0, 0), (0, F - 2)))
    Wr1 = jnp.pad(jnp.transpose(W1, (1, 0, 2)).reshape(2, K * F),
                  ((0, F - 2), (0, 0)))
    r1p = jnp.pad(root1, ((0, F - 2), (0, 0)))
    Wr2 = jnp.transpose(W2, (1, 0, 2)).reshape(F, K * F)
    Wr3 = jnp.transpose(W3, (1, 0, 2)).reshape(F, K * F)

    tab = _mm_call(xp, Wr1)
    h = xp
    for root, bias, Wrn in ((r1p, b1, Wr2), (root2, b2, Wr3)):
        aggf = _layer_call(gflat, bflat, dstc, tab.reshape(N * K, F), zeros)
        h, tab = _lem_call(aggf.reshape(NC, N, F), deg3, h, root,
                           bias.reshape(1, F), Wrn)

    aggf = _layer_call(gflat, bflat, dstc, tab.reshape(N * K, F), zeros)
    out = _lef_call(aggf.reshape(NC, N, F), deg3, h, root3,
                    b3.reshape(1, F), fc_w, fc_b.reshape(1, 1))
    return out[:, 0]


# speedup vs baseline: 17.1993x; 1.8433x over previous
import functools

import jax
import jax.numpy as jnp
from jax import lax
from jax.experimental import pallas as pl
from jax.experimental.pallas import tpu as pltpu
from jax.experimental.pallas import tpu_sc as plsc

N = 10000
E = 160000
K = 25
F = 16

NC, NS = 2, 16
NW = NC * NS
R = E * 4
GR = R // 128
CPW = GR // NW
GREM = GR - CPW * NW
ER = E // 128
RPC = ER // NC
RPT = RPC // NS
NPT = 624
NPT_LAST = N - (NS - 1) * NPT

_mesh = plsc.VectorSubcoreMesh(
    core_axis_name="c", subcore_axis_name="s", num_cores=NC, num_subcores=NS)


def _basis_body(eat_ref, ei_ref, g_ref, b_ref, d_ref):
    ea0 = eat_ref[0, :]
    ea1 = eat_ref[1, :]
    src = ei_ref[0, :]
    v0 = ea0 * 4.0
    v1 = ea1 * 4.0
    lo0 = jnp.floor(v0)
    lo1 = jnp.floor(v1)
    f0 = v0 - lo0
    f1 = v1 - lo1
    i0 = lo0.astype(jnp.int32)
    i1 = lo1.astype(jnp.int32)
    i0b = jnp.minimum(i0 + 1, 4)
    i1b = jnp.minimum(i1 + 1, 4)
    s25 = src * K
    b_ref[...] = jnp.concatenate(
        [((1 - f0) * (1 - f1))[None], ((1 - f0) * f1)[None],
         (f0 * (1 - f1))[None], (f0 * f1)[None]], axis=0)
    g_ref[...] = jnp.concatenate(
        [(s25 + i0 + 5 * i1)[None], (s25 + i0 + 5 * i1b)[None],
         (s25 + i0b + 5 * i1)[None], (s25 + i0b + 5 * i1b)[None]], axis=0)
    d_ref[...] = ei_ref[1:2, :]


_EB = 16000
_NEB = E // _EB
_basis_call = pl.pallas_call(
    _basis_body,
    grid=(_NEB,),
    in_specs=[pl.BlockSpec((2, _EB), lambda i: (0, i)),
              pl.BlockSpec((2, _EB), lambda i: (0, i))],
    out_specs=[pl.BlockSpec((4, _EB), lambda i: (0, i)),
               pl.BlockSpec((4, _EB), lambda i: (0, i)),
               pl.BlockSpec((1, _EB), lambda i: (0, i))],
    out_shape=[jax.ShapeDtypeStruct((4, E), jnp.int32),
               jax.ShapeDtypeStruct((4, E), jnp.float32),
               jax.ShapeDtypeStruct((1, E), jnp.int32)],
)


def _mm_body(h_ref, w_ref, o_ref):
    o_ref[...] = jnp.dot(h_ref[...], w_ref[...],
                         preferred_element_type=jnp.float32)


_NB = 2000
_mm_call = pl.pallas_call(
    _mm_body,
    grid=(N // _NB,),
    in_specs=[pl.BlockSpec((_NB, F), lambda i: (i, 0)),
              pl.BlockSpec((F, K * F), lambda i: (0, 0))],
    out_specs=pl.BlockSpec((_NB, K * F), lambda i: (i, 0)),
    out_shape=jax.ShapeDtypeStruct((N, K * F), jnp.float32),
)


SEG = (RPT + 1) * 128


def _weight_chunk(gbuf, bvals, mbuf, p, b):
    iota = lax.iota(jnp.int32, 16)
    for t in range(8):
        srow = p * 128 + t * 16 + iota
        rows = [p * 512 + j * 128 + t * 16 + iota for j in range(4)]
        bv = [bvals[pl.ds(j * SEG + b * 128 + t * 16, 16)] for j in range(4)]

        @plsc.parallel_loop(0, F, unroll=2)
        def _(i):
            col = lax.bitwise_and(iota + i, F - 1)
            g0 = plsc.load_gather(gbuf, [rows[0], col])
            g1 = plsc.load_gather(gbuf, [rows[1], col])
            g2 = plsc.load_gather(gbuf, [rows[2], col])
            g3 = plsc.load_gather(gbuf, [rows[3], col])
            acc = (bv[0] * g0 + bv[1] * g1) + (bv[2] * g2 + bv[3] * g3)
            plsc.store_scatter(mbuf, [srow, col], acc)


def _layer_body(gflat, bflat, dstc, table, zeros, out,
                idxv, bvals, didx, gbuf, mbuf, acc,
                gsem0, gsem1, ssem0, ssem1):
    c = lax.axis_index("c")
    s = lax.axis_index("s")
    rbase = c * RPC + s * RPT
    ebase = pl.multiple_of(rbase * 128, 128)
    nb = RPT + jnp.where(s == NS - 1, 1, 0)

    for j in range(4):
        pltpu.sync_copy(gflat.at[j, pl.ds(ebase, RPT * 128)],
                        idxv.at[pl.ds(j * SEG, RPT * 128)])
        pltpu.sync_copy(bflat.at[j, pl.ds(ebase, RPT * 128)],
                        bvals.at[pl.ds(j * SEG, RPT * 128)])

    @pl.when(s == NS - 1)
    def _():
        e2 = pl.multiple_of((c * RPC + NS * RPT) * 128, 128)
        for j in range(4):
            pltpu.sync_copy(gflat.at[j, pl.ds(e2, 128)],
                            idxv.at[pl.ds(j * SEG + RPT * 128, 128)])
            pltpu.sync_copy(bflat.at[j, pl.ds(e2, 128)],
                            bvals.at[pl.ds(j * SEG + RPT * 128, 128)])

    _init_acc(zeros, acc, s)
    plsc.subcore_barrier()

    def _fire(b, p, sem):
        for j in range(4):
            pltpu.async_copy(
                table.at[idxv.at[pl.ds(pl.multiple_of(j * SEG + b * 128, 128),
                                       128)]],
                gbuf.at[pl.ds(p * 512 + j * 128, 128)], sem)
        pltpu.async_copy(
            dstc.at[0, pl.ds(pl.multiple_of((rbase + b) * 128, 128), 128)],
            didx.at[lax.rem(b, 4)], sem)

    def _drain(b, p, sem):
        for j in range(4):
            pltpu.make_async_copy(
                table.at[idxv.at[pl.ds(pl.multiple_of(j * SEG + b * 128, 128),
                                       128)]],
                gbuf.at[pl.ds(p * 512 + j * 128, 128)], sem).wait()
        pltpu.make_async_copy(
            dstc.at[0, pl.ds(pl.multiple_of((rbase + b) * 128, 128), 128)],
            didx.at[lax.rem(b, 4)], sem).wait()

    _fire(0, 0, gsem0)

    def step(b, carry):
        p = lax.rem(b, 2)

        def _body(p_lit, gsem, ssem):
            @pl.when(b >= 2)
            def _():
                pltpu.make_async_copy(
                    mbuf.at[pl.ds(p_lit * 128, 128)],
                    acc.at[didx.at[lax.rem(b - 2, 4)]], ssem).wait()

            @pl.when(b + 1 < nb)
            def _():
                _fire(b + 1, 1 - p_lit, gsem1 if p_lit == 0 else gsem0)

            _drain(b, p_lit, gsem)
            _weight_chunk(gbuf, bvals, mbuf, p_lit, b)
            pltpu.async_copy(mbuf.at[pl.ds(p_lit * 128, 128)],
                             acc.at[didx.at[lax.rem(b, 4)]], ssem,
                             add=True)

        @pl.when(p == 0)
        def _():
            _body(0, gsem0, ssem0)

        @pl.when(p == 1)
        def _():
            _body(1, gsem1, ssem1)

        return carry

    lax.fori_loop(0, nb, step, 0)
    pltpu.make_async_copy(mbuf.at[pl.ds(0, 128)],
                          acc.at[didx.at[0]], ssem0).wait()
    pltpu.make_async_copy(mbuf.at[pl.ds(128, 128)],
                          acc.at[didx.at[1]], ssem1).wait()
    plsc.subcore_barrier()
    _copy_out(acc, out, c, s)


_layer_call = functools.partial(
    pl.kernel,
    out_type=jax.ShapeDtypeStruct((NC * N, F), jnp.float32),
    mesh=_mesh,
    compiler_params=pltpu.CompilerParams(use_tc_tiling_on_sc=False, needs_layout_passes=False),
    scratch_types=[pltpu.VMEM((4 * SEG,), jnp.int32),
                   pltpu.VMEM((4 * SEG,), jnp.float32),
                   pltpu.VMEM((4, 128), jnp.int32),
                   pltpu.VMEM((1024, F), jnp.float32),
                   pltpu.VMEM((256, F), jnp.float32),
                   pltpu.VMEM_SHARED((N, F), jnp.float32),
                   pltpu.SemaphoreType.DMA,
                   pltpu.SemaphoreType.DMA,
                   pltpu.SemaphoreType.DMA,
                   pltpu.SemaphoreType.DMA],
)(_layer_body)


def _init_acc(zeros, acc, s):
    nb = pl.multiple_of(s * NPT, 8)

    @pl.when(s < NS - 1)
    def _():
        pltpu.sync_copy(zeros.at[pl.ds(nb, NPT)], acc.at[pl.ds(nb, NPT)])

    @pl.when(s == NS - 1)
    def _():
        b0 = (NS - 1) * NPT
        pltpu.sync_copy(zeros.at[pl.ds(b0, NPT_LAST)],
                        acc.at[pl.ds(b0, NPT_LAST)])


def _copy_out(acc, out, c, s):
    nb = pl.multiple_of(s * NPT, 8)

    @pl.when(s < NS - 1)
    def _():
        pltpu.sync_copy(acc.at[pl.ds(nb, NPT)],
                        out.at[pl.ds(pl.multiple_of(c * N + nb, 8), NPT)])

    @pl.when(s == NS - 1)
    def _():
        b0 = (NS - 1) * NPT
        pltpu.sync_copy(
            acc.at[pl.ds(b0, NPT_LAST)],
            out.at[pl.ds(pl.multiple_of(c * N + b0, 8), NPT_LAST)])


def _deg_body(dst1, ones128, zeros, out, didx, obuf, acc, msem):
    c = lax.axis_index("c")
    s = lax.axis_index("s")
    _init_acc(zeros, acc, s)
    pltpu.sync_copy(ones128, obuf)
    rbase = c * RPC + s * RPT
    plsc.subcore_barrier()

    def chunk(b, carry):
        off = pl.multiple_of((rbase + b) * 128, 128)
        pltpu.sync_copy(dst1.at[0, pl.ds(off, 128)], didx.at[0])
        pltpu.sync_copy(obuf, acc.at[didx.at[0]], add=True)
        return carry

    lax.fori_loop(0, RPT, chunk, 0)

    @pl.when(s == NS - 1)
    def _():
        off = pl.multiple_of((c * RPC + NS * RPT) * 128, 128)
        pltpu.sync_copy(dst1.at[0, pl.ds(off, 128)], didx.at[0])
        pltpu.sync_copy(obuf, acc.at[didx.at[0]], add=True)

    plsc.subcore_barrier()
    _copy_out(acc, out, c, s)


_deg_call = functools.partial(
    pl.kernel,
    out_type=jax.ShapeDtypeStruct((NC * N, F), jnp.float32),
    mesh=_mesh,
    compiler_params=pltpu.CompilerParams(use_tc_tiling_on_sc=False, needs_layout_passes=False),
    scratch_types=[pltpu.VMEM((1, 128), jnp.int32),
                   pltpu.VMEM((128, F), jnp.float32),
                   pltpu.VMEM_SHARED((N, F), jnp.float32),
                   pltpu.SemaphoreType.DMA],
)(_deg_body)


def _relu_layer(agg_ref, deg_ref, h_ref, root_ref, bias_ref):
    a = agg_ref[0] + agg_ref[1]
    deg = deg_ref[0, :, 0:1] + deg_ref[1, :, 0:1]
    a = a / jnp.maximum(deg, 1.0)
    return jax.nn.relu(
        a + jnp.dot(h_ref[...], root_ref[...],
                    preferred_element_type=jnp.float32) + bias_ref[...])


def _lem_body(agg_ref, deg_ref, h_ref, root_ref, bias_ref, wr_ref,
              hn_ref, tab_ref):
    hn = _relu_layer(agg_ref, deg_ref, h_ref, root_ref, bias_ref)
    hn_ref[...] = hn
    tab_ref[...] = jnp.dot(hn, wr_ref[...],
                           preferred_element_type=jnp.float32)


_lem_call = pl.pallas_call(
    _lem_body,
    grid=(N // _NB,),
    in_specs=[pl.BlockSpec((NC, _NB, F), lambda i: (0, i, 0)),
              pl.BlockSpec((NC, _NB, F), lambda i: (0, i, 0)),
              pl.BlockSpec((_NB, F), lambda i: (i, 0)),
              pl.BlockSpec((F, F), lambda i: (0, 0)),
              pl.BlockSpec((1, F), lambda i: (0, 0)),
              pl.BlockSpec((F, K * F), lambda i: (0, 0))],
    out_specs=[pl.BlockSpec((_NB, F), lambda i: (i, 0)),
               pl.BlockSpec((_NB, K * F), lambda i: (i, 0))],
    out_shape=[jax.ShapeDtypeStruct((N, F), jnp.float32),
               jax.ShapeDtypeStruct((N, K * F), jnp.float32)],
)


def _lef_body(agg_ref, deg_ref, h_ref, root_ref, bias_ref, fw_ref, fb_ref,
              o_ref):
    hn = _relu_layer(agg_ref, deg_ref, h_ref, root_ref, bias_ref)
    o_ref[...] = jax.nn.sigmoid(
        jnp.dot(hn, fw_ref[...],
                preferred_element_type=jnp.float32) + fb_ref[...])


_lef_call = pl.pallas_call(
    _lef_body,
    grid=(N // _NB,),
    in_specs=[pl.BlockSpec((NC, _NB, F), lambda i: (0, i, 0)),
              pl.BlockSpec((NC, _NB, F), lambda i: (0, i, 0)),
              pl.BlockSpec((_NB, F), lambda i: (i, 0)),
              pl.BlockSpec((F, F), lambda i: (0, 0)),
              pl.BlockSpec((1, F), lambda i: (0, 0)),
              pl.BlockSpec((F, 1), lambda i: (0, 0)),
              pl.BlockSpec((1, 1), lambda i: (0, 0))],
    out_specs=pl.BlockSpec((_NB, 1), lambda i: (i, 0)),
    out_shape=jax.ShapeDtypeStruct((N, 1), jnp.float32),
)


def kernel(x, edge_index, edge_attr, W1, root1, b1, W2, root2, b2,
           W3, root3, b3, fc_w, fc_b):
    f32 = jnp.float32
    eat = edge_attr.T

    gflat, bflat, dstc = _basis_call(eat, edge_index)

    zeros = jnp.zeros((N, F), f32)
    ones128 = jnp.ones((128, F), f32)
    deg3 = _deg_call(dstc, ones128, zeros).reshape(NC, N, F)

    xp = jnp.pad(x, ((0, 0), (0, F - 2)))
    Wr1 = jnp.pad(jnp.transpose(W1, (1, 0, 2)).reshape(2, K * F),
                  ((0, F - 2), (0, 0)))
    r1p = jnp.pad(root1, ((0, F - 2), (0, 0)))
    Wr2 = jnp.transpose(W2, (1, 0, 2)).reshape(F, K * F)
    Wr3 = jnp.transpose(W3, (1, 0, 2)).reshape(F, K * F)

    tab = _mm_call(xp, Wr1)
    h = xp
    for root, bias, Wrn in ((r1p, b1, Wr2), (root2, b2, Wr3)):
        aggf = _layer_call(gflat, bflat, dstc, tab.reshape(N * K, F), zeros)
        h, tab = _lem_call(aggf.reshape(NC, N, F), deg3, h, root,
                           bias.reshape(1, F), Wrn)

    aggf = _layer_call(gflat, bflat, dstc, tab.reshape(N * K, F), zeros)
    out = _lef_call(aggf.reshape(NC, N, F), deg3, h, root3,
                    b3.reshape(1, F), fc_w, fc_b.reshape(1, 1))
    return out[:, 0]


# speedup vs baseline: 19.3866x; 1.1272x over previous
import functools

import jax
import jax.numpy as jnp
from jax import lax
from jax.experimental import pallas as pl
from jax.experimental.pallas import tpu as pltpu
from jax.experimental.pallas import tpu_sc as plsc

N = 10000
E = 160000
K = 25
F = 16

NC, NS = 2, 16
NW = NC * NS
R = E * 4
GR = R // 128
CPW = GR // NW
GREM = GR - CPW * NW
ER = E // 128
RPC = ER // NC
RPT = RPC // NS
NPT = 624
NPT_LAST = N - (NS - 1) * NPT

_mesh = plsc.VectorSubcoreMesh(
    core_axis_name="c", subcore_axis_name="s", num_cores=NC, num_subcores=NS)


def _basis_body(eat_ref, ei_ref, g_ref, b_ref, d_ref):
    ea0 = eat_ref[0, :]
    ea1 = eat_ref[1, :]
    src = ei_ref[0, :]
    v0 = ea0 * 4.0
    v1 = ea1 * 4.0
    lo0 = jnp.floor(v0)
    lo1 = jnp.floor(v1)
    f0 = v0 - lo0
    f1 = v1 - lo1
    i0 = lo0.astype(jnp.int32)
    i1 = lo1.astype(jnp.int32)
    i0b = jnp.minimum(i0 + 1, 4)
    i1b = jnp.minimum(i1 + 1, 4)
    s25 = src * K
    b_ref[...] = jnp.concatenate(
        [((1 - f0) * (1 - f1))[None], ((1 - f0) * f1)[None],
         (f0 * (1 - f1))[None], (f0 * f1)[None]], axis=0)
    g_ref[...] = jnp.concatenate(
        [(s25 + i0 + 5 * i1)[None], (s25 + i0 + 5 * i1b)[None],
         (s25 + i0b + 5 * i1)[None], (s25 + i0b + 5 * i1b)[None]], axis=0)
    d_ref[...] = ei_ref[1:2, :]


_EB = 16000
_NEB = E // _EB
_basis_call = pl.pallas_call(
    _basis_body,
    grid=(_NEB,),
    in_specs=[pl.BlockSpec((2, _EB), lambda i: (0, i)),
              pl.BlockSpec((2, _EB), lambda i: (0, i))],
    out_specs=[pl.BlockSpec((4, _EB), lambda i: (0, i)),
               pl.BlockSpec((4, _EB), lambda i: (0, i)),
               pl.BlockSpec((1, _EB), lambda i: (0, i))],
    out_shape=[jax.ShapeDtypeStruct((4, E), jnp.int32),
               jax.ShapeDtypeStruct((4, E), jnp.float32),
               jax.ShapeDtypeStruct((1, E), jnp.int32)],
)


def _mm_body(h_ref, w_ref, o_ref):
    o_ref[...] = jnp.dot(h_ref[...], w_ref[...],
                         preferred_element_type=jnp.float32)


_NB = 2000
_mm_call = pl.pallas_call(
    _mm_body,
    grid=(N // _NB,),
    in_specs=[pl.BlockSpec((_NB, F), lambda i: (i, 0)),
              pl.BlockSpec((F, K * F), lambda i: (0, 0))],
    out_specs=pl.BlockSpec((_NB, K * F), lambda i: (i, 0)),
    out_shape=jax.ShapeDtypeStruct((N, K * F), jnp.float32),
)


SEG = (RPT + 1) * 128


def _weight_chunk(gbuf, bvals, mbuf, p, b):
    iota = lax.iota(jnp.int32, 16)
    for t in range(8):
        srow = p * 128 + t * 16 + iota
        rows = [p * 512 + j * 128 + t * 16 + iota for j in range(4)]
        bv = [bvals[pl.ds(j * SEG + b * 128 + t * 16, 16)] for j in range(4)]

        @plsc.parallel_loop(0, F, unroll=2)
        def _(i):
            col = lax.bitwise_and(iota + i, F - 1)
            g0 = plsc.load_gather(gbuf, [rows[0], col])
            g1 = plsc.load_gather(gbuf, [rows[1], col])
            g2 = plsc.load_gather(gbuf, [rows[2], col])
            g3 = plsc.load_gather(gbuf, [rows[3], col])
            acc = (bv[0] * g0 + bv[1] * g1) + (bv[2] * g2 + bv[3] * g3)
            plsc.store_scatter(mbuf, [srow, col], acc)


def _layer_impl(gflat, bflat, dstc, table, zeros, out,
                idxv, bvals, didx, gbuf, mbuf, acc,
                gsem0, gsem1, ssem0, ssem1,
                ones128=None, degout=None, obuf=None, acc2=None):
    do_deg = degout is not None
    c = lax.axis_index("c")
    s = lax.axis_index("s")
    rbase = c * RPC + s * RPT
    ebase = pl.multiple_of(rbase * 128, 128)
    nb = RPT + jnp.where(s == NS - 1, 1, 0)

    cps = []
    for j in range(4):
        cps.append(pltpu.async_copy(gflat.at[j, pl.ds(ebase, RPT * 128)],
                                    idxv.at[pl.ds(j * SEG, RPT * 128)],
                                    gsem0))
        cps.append(pltpu.async_copy(bflat.at[j, pl.ds(ebase, RPT * 128)],
                                    bvals.at[pl.ds(j * SEG, RPT * 128)],
                                    gsem1))

    @pl.when(s == NS - 1)
    def _():
        e2 = pl.multiple_of((c * RPC + NS * RPT) * 128, 128)
        tcps = []
        for j in range(4):
            tcps.append(pltpu.async_copy(
                gflat.at[j, pl.ds(e2, 128)],
                idxv.at[pl.ds(j * SEG + RPT * 128, 128)], gsem0))
            tcps.append(pltpu.async_copy(
                bflat.at[j, pl.ds(e2, 128)],
                bvals.at[pl.ds(j * SEG + RPT * 128, 128)], gsem1))
        for cp in tcps:
            cp.wait()

    if do_deg:
        pltpu.sync_copy(ones128, obuf)
        _init_acc(zeros, acc2, s)
    _init_acc(zeros, acc, s)
    for cp in cps:
        cp.wait()
    plsc.subcore_barrier()

    def _fire(b, p, sem):
        for j in range(4):
            pltpu.async_copy(
                table.at[idxv.at[pl.ds(pl.multiple_of(j * SEG + b * 128, 128),
                                       128)]],
                gbuf.at[pl.ds(p * 512 + j * 128, 128)], sem)
        pltpu.async_copy(
            dstc.at[0, pl.ds(pl.multiple_of((rbase + b) * 128, 128), 128)],
            didx.at[lax.rem(b, 4)], sem)

    def _drain(b, p, sem):
        for j in range(4):
            pltpu.make_async_copy(
                table.at[idxv.at[pl.ds(pl.multiple_of(j * SEG + b * 128, 128),
                                       128)]],
                gbuf.at[pl.ds(p * 512 + j * 128, 128)], sem).wait()
        pltpu.make_async_copy(
            dstc.at[0, pl.ds(pl.multiple_of((rbase + b) * 128, 128), 128)],
            didx.at[lax.rem(b, 4)], sem).wait()

    _fire(0, 0, gsem0)

    def step(b, carry):
        p = lax.rem(b, 2)

        def _body(p_lit, gsem, ssem):
            @pl.when(b >= 2)
            def _():
                pltpu.make_async_copy(
                    mbuf.at[pl.ds(p_lit * 128, 128)],
                    acc.at[didx.at[lax.rem(b - 2, 4)]], ssem).wait()
                if do_deg:
                    pltpu.make_async_copy(
                        obuf, acc2.at[didx.at[lax.rem(b - 2, 4)]],
                        ssem).wait()

            @pl.when(b + 1 < nb)
            def _():
                _fire(b + 1, 1 - p_lit, gsem1 if p_lit == 0 else gsem0)

            _drain(b, p_lit, gsem)
            _weight_chunk(gbuf, bvals, mbuf, p_lit, b)
            pltpu.async_copy(mbuf.at[pl.ds(p_lit * 128, 128)],
                             acc.at[didx.at[lax.rem(b, 4)]], ssem,
                             add=True)
            if do_deg:
                pltpu.async_copy(obuf, acc2.at[didx.at[lax.rem(b, 4)]],
                                 ssem, add=True)

        @pl.when(p == 0)
        def _():
            _body(0, gsem0, ssem0)

        @pl.when(p == 1)
        def _():
            _body(1, gsem1, ssem1)

        return carry

    lax.fori_loop(0, nb, step, 0)
    pltpu.make_async_copy(mbuf.at[pl.ds(0, 128)],
                          acc.at[didx.at[0]], ssem0).wait()
    pltpu.make_async_copy(mbuf.at[pl.ds(128, 128)],
                          acc.at[didx.at[1]], ssem1).wait()
    if do_deg:
        pltpu.make_async_copy(obuf, acc2.at[didx.at[0]], ssem0).wait()
        pltpu.make_async_copy(obuf, acc2.at[didx.at[1]], ssem1).wait()
    plsc.subcore_barrier()
    _copy_out(acc, out, c, s)
    if do_deg:
        _copy_out(acc2, degout, c, s)


def _layer_body(gflat, bflat, dstc, table, zeros, out,
                idxv, bvals, didx, gbuf, mbuf, acc,
                gsem0, gsem1, ssem0, ssem1):
    _layer_impl(gflat, bflat, dstc, table, zeros, out,
                idxv, bvals, didx, gbuf, mbuf, acc,
                gsem0, gsem1, ssem0, ssem1)


def _layer1_body(gflat, bflat, dstc, table, zeros, ones128, out, degout,
                 idxv, bvals, didx, gbuf, mbuf, obuf, acc, acc2,
                 gsem0, gsem1, ssem0, ssem1):
    _layer_impl(gflat, bflat, dstc, table, zeros, out,
                idxv, bvals, didx, gbuf, mbuf, acc,
                gsem0, gsem1, ssem0, ssem1,
                ones128=ones128, degout=degout, obuf=obuf, acc2=acc2)


_SC_PARAMS = pltpu.CompilerParams(use_tc_tiling_on_sc=False,
                                  needs_layout_passes=False)
_SC_SCRATCH = [pltpu.VMEM((4 * SEG,), jnp.int32),
               pltpu.VMEM((4 * SEG,), jnp.float32),
               pltpu.VMEM((4, 128), jnp.int32),
               pltpu.VMEM((1024, F), jnp.float32),
               pltpu.VMEM((256, F), jnp.float32),
               pltpu.VMEM_SHARED((N, F), jnp.float32),
               pltpu.SemaphoreType.DMA,
               pltpu.SemaphoreType.DMA,
               pltpu.SemaphoreType.DMA,
               pltpu.SemaphoreType.DMA]

_layer_call = functools.partial(
    pl.kernel,
    out_type=jax.ShapeDtypeStruct((NC * N, F), jnp.float32),
    mesh=_mesh,
    compiler_params=_SC_PARAMS,
    scratch_types=_SC_SCRATCH,
)(_layer_body)

_layer1_call = functools.partial(
    pl.kernel,
    out_type=[jax.ShapeDtypeStruct((NC * N, F), jnp.float32),
              jax.ShapeDtypeStruct((NC * N, F), jnp.float32)],
    mesh=_mesh,
    compiler_params=_SC_PARAMS,
    scratch_types=[pltpu.VMEM((4 * SEG,), jnp.int32),
                   pltpu.VMEM((4 * SEG,), jnp.float32),
                   pltpu.VMEM((4, 128), jnp.int32),
                   pltpu.VMEM((1024, F), jnp.float32),
                   pltpu.VMEM((256, F), jnp.float32),
                   pltpu.VMEM((128, F), jnp.float32),
                   pltpu.VMEM_SHARED((N, F), jnp.float32),
                   pltpu.VMEM_SHARED((N, F), jnp.float32),
                   pltpu.SemaphoreType.DMA,
                   pltpu.SemaphoreType.DMA,
                   pltpu.SemaphoreType.DMA,
                   pltpu.SemaphoreType.DMA],
)(_layer1_body)


def _init_acc(zeros, acc, s):
    nb = pl.multiple_of(s * NPT, 8)

    @pl.when(s < NS - 1)
    def _():
        pltpu.sync_copy(zeros.at[pl.ds(nb, NPT)], acc.at[pl.ds(nb, NPT)])

    @pl.when(s == NS - 1)
    def _():
        b0 = (NS - 1) * NPT
        pltpu.sync_copy(zeros.at[pl.ds(b0, NPT_LAST)],
                        acc.at[pl.ds(b0, NPT_LAST)])


def _copy_out(acc, out, c, s):
    nb = pl.multiple_of(s * NPT, 8)

    @pl.when(s < NS - 1)
    def _():
        pltpu.sync_copy(acc.at[pl.ds(nb, NPT)],
                        out.at[pl.ds(pl.multiple_of(c * N + nb, 8), NPT)])

    @pl.when(s == NS - 1)
    def _():
        b0 = (NS - 1) * NPT
        pltpu.sync_copy(
            acc.at[pl.ds(b0, NPT_LAST)],
            out.at[pl.ds(pl.multiple_of(c * N + b0, 8), NPT_LAST)])


def _relu_layer(agg_ref, deg_ref, h_ref, root_ref, bias_ref):
    a = agg_ref[0] + agg_ref[1]
    deg = deg_ref[0, :, 0:1] + deg_ref[1, :, 0:1]
    a = a / jnp.maximum(deg, 1.0)
    return jax.nn.relu(
        a + jnp.dot(h_ref[...], root_ref[...],
                    preferred_element_type=jnp.float32) + bias_ref[...])


def _lem_body(agg_ref, deg_ref, h_ref, root_ref, bias_ref, wr_ref,
              hn_ref, tab_ref):
    hn = _relu_layer(agg_ref, deg_ref, h_ref, root_ref, bias_ref)
    hn_ref[...] = hn
    tab_ref[...] = jnp.dot(hn, wr_ref[...],
                           preferred_element_type=jnp.float32)


_lem_call = pl.pallas_call(
    _lem_body,
    grid=(N // _NB,),
    in_specs=[pl.BlockSpec((NC, _NB, F), lambda i: (0, i, 0)),
              pl.BlockSpec((NC, _NB, F), lambda i: (0, i, 0)),
              pl.BlockSpec((_NB, F), lambda i: (i, 0)),
              pl.BlockSpec((F, F), lambda i: (0, 0)),
              pl.BlockSpec((1, F), lambda i: (0, 0)),
              pl.BlockSpec((F, K * F), lambda i: (0, 0))],
    out_specs=[pl.BlockSpec((_NB, F), lambda i: (i, 0)),
               pl.BlockSpec((_NB, K * F), lambda i: (i, 0))],
    out_shape=[jax.ShapeDtypeStruct((N, F), jnp.float32),
               jax.ShapeDtypeStruct((N, K * F), jnp.float32)],
)


def _lef_body(agg_ref, deg_ref, h_ref, root_ref, bias_ref, fw_ref, fb_ref,
              o_ref):
    hn = _relu_layer(agg_ref, deg_ref, h_ref, root_ref, bias_ref)
    o_ref[...] = jax.nn.sigmoid(
        jnp.dot(hn, fw_ref[...],
                preferred_element_type=jnp.float32) + fb_ref[...])


_lef_call = pl.pallas_call(
    _lef_body,
    grid=(N // _NB,),
    in_specs=[pl.BlockSpec((NC, _NB, F), lambda i: (0, i, 0)),
              pl.BlockSpec((NC, _NB, F), lambda i: (0, i, 0)),
              pl.BlockSpec((_NB, F), lambda i: (i, 0)),
              pl.BlockSpec((F, F), lambda i: (0, 0)),
              pl.BlockSpec((1, F), lambda i: (0, 0)),
              pl.BlockSpec((F, 1), lambda i: (0, 0)),
              pl.BlockSpec((1, 1), lambda i: (0, 0))],
    out_specs=pl.BlockSpec((_NB, 1), lambda i: (i, 0)),
    out_shape=jax.ShapeDtypeStruct((N, 1), jnp.float32),
)


def kernel(x, edge_index, edge_attr, W1, root1, b1, W2, root2, b2,
           W3, root3, b3, fc_w, fc_b):
    f32 = jnp.float32
    eat = edge_attr.T

    gflat, bflat, dstc = _basis_call(eat, edge_index)

    zeros = jnp.zeros((N, F), f32)
    ones128 = jnp.ones((128, F), f32)

    xp = jnp.pad(x, ((0, 0), (0, F - 2)))
    Wr1 = jnp.pad(jnp.transpose(W1, (1, 0, 2)).reshape(2, K * F),
                  ((0, F - 2), (0, 0)))
    r1p = jnp.pad(root1, ((0, F - 2), (0, 0)))
    Wr2 = jnp.transpose(W2, (1, 0, 2)).reshape(F, K * F)
    Wr3 = jnp.transpose(W3, (1, 0, 2)).reshape(F, K * F)

    tab = _mm_call(xp, Wr1)
    h = xp
    aggf, degf = _layer1_call(gflat, bflat, dstc, tab.reshape(N * K, F),
                              zeros, ones128)
    deg3 = degf.reshape(NC, N, F)
    h, tab = _lem_call(aggf.reshape(NC, N, F), deg3, h, r1p,
                       b1.reshape(1, F), Wr2)
    aggf = _layer_call(gflat, bflat, dstc, tab.reshape(N * K, F), zeros)
    h, tab = _lem_call(aggf.reshape(NC, N, F), deg3, h, root2,
                       b2.reshape(1, F), Wr3)

    aggf = _layer_call(gflat, bflat, dstc, tab.reshape(N * K, F), zeros)
    out = _lef_call(aggf.reshape(NC, N, F), deg3, h, root3,
                    b3.reshape(1, F), fc_w, fc_b.reshape(1, 1))
    return out[:, 0]


# speedup vs baseline: 19.5492x; 1.0084x over previous
import functools

import jax
import jax.numpy as jnp
from jax import lax
from jax.experimental import pallas as pl
from jax.experimental.pallas import tpu as pltpu
from jax.experimental.pallas import tpu_sc as plsc

N = 10000
E = 160000
K = 25
F = 16

NC, NS = 2, 16
NW = NC * NS
R = E * 4
GR = R // 128
CPW = GR // NW
GREM = GR - CPW * NW
ER = E // 128
RPC = ER // NC
RPT = RPC // NS
NPT = 624
NPT_LAST = N - (NS - 1) * NPT

_mesh = plsc.VectorSubcoreMesh(
    core_axis_name="c", subcore_axis_name="s", num_cores=NC, num_subcores=NS)


def _basis_body(eat_ref, ei_ref, g_ref, b_ref, d_ref):
    ea0 = eat_ref[0, :]
    ea1 = eat_ref[1, :]
    src = ei_ref[0, :]
    v0 = ea0 * 4.0
    v1 = ea1 * 4.0
    lo0 = jnp.floor(v0)
    lo1 = jnp.floor(v1)
    f0 = v0 - lo0
    f1 = v1 - lo1
    i0 = lo0.astype(jnp.int32)
    i1 = lo1.astype(jnp.int32)
    i0b = jnp.minimum(i0 + 1, 4)
    i1b = jnp.minimum(i1 + 1, 4)
    s25 = src * K
    b_ref[...] = jnp.concatenate(
        [((1 - f0) * (1 - f1))[None], ((1 - f0) * f1)[None],
         (f0 * (1 - f1))[None], (f0 * f1)[None]], axis=0)
    g_ref[...] = jnp.concatenate(
        [(s25 + i0 + 5 * i1)[None], (s25 + i0 + 5 * i1b)[None],
         (s25 + i0b + 5 * i1)[None], (s25 + i0b + 5 * i1b)[None]], axis=0)
    d_ref[...] = ei_ref[1:2, :]


_EB = 16000
_NEB = E // _EB
_basis_call = pl.pallas_call(
    _basis_body,
    grid=(_NEB,),
    in_specs=[pl.BlockSpec((2, _EB), lambda i: (0, i)),
              pl.BlockSpec((2, _EB), lambda i: (0, i))],
    out_specs=[pl.BlockSpec((4, _EB), lambda i: (0, i)),
               pl.BlockSpec((4, _EB), lambda i: (0, i)),
               pl.BlockSpec((1, _EB), lambda i: (0, i))],
    out_shape=[jax.ShapeDtypeStruct((4, E), jnp.int32),
               jax.ShapeDtypeStruct((4, E), jnp.float32),
               jax.ShapeDtypeStruct((1, E), jnp.int32)],
)


def _mm_body(h_ref, w_ref, o_ref):
    o_ref[...] = jnp.dot(h_ref[...], w_ref[...],
                         preferred_element_type=jnp.float32)


_NB = 2000
_mm_call = pl.pallas_call(
    _mm_body,
    grid=(N // _NB,),
    in_specs=[pl.BlockSpec((_NB, F), lambda i: (i, 0)),
              pl.BlockSpec((F, K * F), lambda i: (0, 0))],
    out_specs=pl.BlockSpec((_NB, K * F), lambda i: (i, 0)),
    out_shape=jax.ShapeDtypeStruct((N, K * F), jnp.float32),
)


SEG = (RPT + 1) * 128

_GATHER_DNUMS = lax.GatherDimensionNumbers(
    offset_dims=(), collapsed_slice_dims=(0,), start_index_map=(0,))


def _bcast_lane(vec, lane):
    return lax.gather(vec, lane[:, None], _GATHER_DNUMS, slice_sizes=(1,),
                      mode=lax.GatherScatterMode.PROMISE_IN_BOUNDS)


def _weight_chunk(gbuf, bvals, mbuf, p, b):
    for t in range(8):
        bv = [bvals[pl.ds(j * SEG + b * 128 + t * 16, 16)] for j in range(4)]

        @plsc.parallel_loop(0, 16, unroll=2)
        def _(l):
            base = p * 512 + t * 16 + l
            lane = jnp.full((16,), l, jnp.int32)
            g0 = gbuf[base, :]
            g1 = gbuf[base + 128, :]
            g2 = gbuf[base + 256, :]
            g3 = gbuf[base + 384, :]
            w0 = _bcast_lane(bv[0], lane)
            w1 = _bcast_lane(bv[1], lane)
            w2 = _bcast_lane(bv[2], lane)
            w3 = _bcast_lane(bv[3], lane)
            mbuf[p * 128 + t * 16 + l, :] = (
                (w0 * g0 + w1 * g1) + (w2 * g2 + w3 * g3))


def _layer_impl(gflat, bflat, dstc, table, zeros, out,
                idxv, bvals, didx, gbuf, mbuf, acc,
                gsem0, gsem1, ssem0, ssem1,
                ones128=None, degout=None, obuf=None, acc2=None):
    do_deg = degout is not None
    c = lax.axis_index("c")
    s = lax.axis_index("s")
    rbase = c * RPC + s * RPT
    ebase = pl.multiple_of(rbase * 128, 128)
    nb = RPT + jnp.where(s == NS - 1, 1, 0)

    cps = []
    for j in range(4):
        cps.append(pltpu.async_copy(gflat.at[j, pl.ds(ebase, RPT * 128)],
                                    idxv.at[pl.ds(j * SEG, RPT * 128)],
                                    gsem0))
        cps.append(pltpu.async_copy(bflat.at[j, pl.ds(ebase, RPT * 128)],
                                    bvals.at[pl.ds(j * SEG, RPT * 128)],
                                    gsem1))

    @pl.when(s == NS - 1)
    def _():
        e2 = pl.multiple_of((c * RPC + NS * RPT) * 128, 128)
        tcps = []
        for j in range(4):
            tcps.append(pltpu.async_copy(
                gflat.at[j, pl.ds(e2, 128)],
                idxv.at[pl.ds(j * SEG + RPT * 128, 128)], gsem0))
            tcps.append(pltpu.async_copy(
                bflat.at[j, pl.ds(e2, 128)],
                bvals.at[pl.ds(j * SEG + RPT * 128, 128)], gsem1))
        for cp in tcps:
            cp.wait()

    if do_deg:
        pltpu.sync_copy(ones128, obuf)
        _init_acc(zeros, acc2, s)
    _init_acc(zeros, acc, s)
    for cp in cps:
        cp.wait()
    plsc.subcore_barrier()

    def _fire(b, p, sem):
        for j in range(4):
            pltpu.async_copy(
                table.at[idxv.at[pl.ds(pl.multiple_of(j * SEG + b * 128, 128),
                                       128)]],
                gbuf.at[pl.ds(p * 512 + j * 128, 128)], sem)
        pltpu.async_copy(
            dstc.at[0, pl.ds(pl.multiple_of((rbase + b) * 128, 128), 128)],
            didx.at[lax.rem(b, 4)], sem)

    def _drain(b, p, sem):
        for j in range(4):
            pltpu.make_async_copy(
                table.at[idxv.at[pl.ds(pl.multiple_of(j * SEG + b * 128, 128),
                                       128)]],
                gbuf.at[pl.ds(p * 512 + j * 128, 128)], sem).wait()
        pltpu.make_async_copy(
            dstc.at[0, pl.ds(pl.multiple_of((rbase + b) * 128, 128), 128)],
            didx.at[lax.rem(b, 4)], sem).wait()

    _fire(0, 0, gsem0)

    def step(b, carry):
        p = lax.rem(b, 2)

        def _body(p_lit, gsem, ssem):
            @pl.when(b >= 2)
            def _():
                pltpu.make_async_copy(
                    mbuf.at[pl.ds(p_lit * 128, 128)],
                    acc.at[didx.at[lax.rem(b - 2, 4)]], ssem).wait()
                if do_deg:
                    pltpu.make_async_copy(
                        obuf, acc2.at[didx.at[lax.rem(b - 2, 4)]],
                        ssem).wait()

            @pl.when(b + 1 < nb)
            def _():
                _fire(b + 1, 1 - p_lit, gsem1 if p_lit == 0 else gsem0)

            _drain(b, p_lit, gsem)
            _weight_chunk(gbuf, bvals, mbuf, p_lit, b)
            pltpu.async_copy(mbuf.at[pl.ds(p_lit * 128, 128)],
                             acc.at[didx.at[lax.rem(b, 4)]], ssem,
                             add=True)
            if do_deg:
                pltpu.async_copy(obuf, acc2.at[didx.at[lax.rem(b, 4)]],
                                 ssem, add=True)

        @pl.when(p == 0)
        def _():
            _body(0, gsem0, ssem0)

        @pl.when(p == 1)
        def _():
            _body(1, gsem1, ssem1)

        return carry

    lax.fori_loop(0, nb, step, 0)
    pltpu.make_async_copy(mbuf.at[pl.ds(0, 128)],
                          acc.at[didx.at[0]], ssem0).wait()
    pltpu.make_async_copy(mbuf.at[pl.ds(128, 128)],
                          acc.at[didx.at[1]], ssem1).wait()
    if do_deg:
        pltpu.make_async_copy(obuf, acc2.at[didx.at[0]], ssem0).wait()
        pltpu.make_async_copy(obuf, acc2.at[didx.at[1]], ssem1).wait()
    plsc.subcore_barrier()
    _copy_out(acc, out, c, s)
    if do_deg:
        _copy_out(acc2, degout, c, s)


def _layer_body(gflat, bflat, dstc, table, zeros, out,
                idxv, bvals, didx, gbuf, mbuf, acc,
                gsem0, gsem1, ssem0, ssem1):
    _layer_impl(gflat, bflat, dstc, table, zeros, out,
                idxv, bvals, didx, gbuf, mbuf, acc,
                gsem0, gsem1, ssem0, ssem1)


def _layer1_body(gflat, bflat, dstc, table, zeros, ones128, out, degout,
                 idxv, bvals, didx, gbuf, mbuf, obuf, acc, acc2,
                 gsem0, gsem1, ssem0, ssem1):
    _layer_impl(gflat, bflat, dstc, table, zeros, out,
                idxv, bvals, didx, gbuf, mbuf, acc,
                gsem0, gsem1, ssem0, ssem1,
                ones128=ones128, degout=degout, obuf=obuf, acc2=acc2)


_SC_PARAMS = pltpu.CompilerParams(use_tc_tiling_on_sc=False,
                                  needs_layout_passes=False)
_SC_SCRATCH = [pltpu.VMEM((4 * SEG,), jnp.int32),
               pltpu.VMEM((4 * SEG,), jnp.float32),
               pltpu.VMEM((4, 128), jnp.int32),
               pltpu.VMEM((1024, F), jnp.float32),
               pltpu.VMEM((256, F), jnp.float32),
               pltpu.VMEM_SHARED((N, F), jnp.float32),
               pltpu.SemaphoreType.DMA,
               pltpu.SemaphoreType.DMA,
               pltpu.SemaphoreType.DMA,
               pltpu.SemaphoreType.DMA]

_layer_call = functools.partial(
    pl.kernel,
    out_type=jax.ShapeDtypeStruct((NC * N, F), jnp.float32),
    mesh=_mesh,
    compiler_params=_SC_PARAMS,
    scratch_types=_SC_SCRATCH,
)(_layer_body)

_layer1_call = functools.partial(
    pl.kernel,
    out_type=[jax.ShapeDtypeStruct((NC * N, F), jnp.float32),
              jax.ShapeDtypeStruct((NC * N, F), jnp.float32)],
    mesh=_mesh,
    compiler_params=_SC_PARAMS,
    scratch_types=[pltpu.VMEM((4 * SEG,), jnp.int32),
                   pltpu.VMEM((4 * SEG,), jnp.float32),
                   pltpu.VMEM((4, 128), jnp.int32),
                   pltpu.VMEM((1024, F), jnp.float32),
                   pltpu.VMEM((256, F), jnp.float32),
                   pltpu.VMEM((128, F), jnp.float32),
                   pltpu.VMEM_SHARED((N, F), jnp.float32),
                   pltpu.VMEM_SHARED((N, F), jnp.float32),
                   pltpu.SemaphoreType.DMA,
                   pltpu.SemaphoreType.DMA,
                   pltpu.SemaphoreType.DMA,
                   pltpu.SemaphoreType.DMA],
)(_layer1_body)


def _init_acc(zeros, acc, s):
    nb = pl.multiple_of(s * NPT, 8)

    @pl.when(s < NS - 1)
    def _():
        pltpu.sync_copy(zeros.at[pl.ds(nb, NPT)], acc.at[pl.ds(nb, NPT)])

    @pl.when(s == NS - 1)
    def _():
        b0 = (NS - 1) * NPT
        pltpu.sync_copy(zeros.at[pl.ds(b0, NPT_LAST)],
                        acc.at[pl.ds(b0, NPT_LAST)])


def _copy_out(acc, out, c, s):
    nb = pl.multiple_of(s * NPT, 8)

    @pl.when(s < NS - 1)
    def _():
        pltpu.sync_copy(acc.at[pl.ds(nb, NPT)],
                        out.at[pl.ds(pl.multiple_of(c * N + nb, 8), NPT)])

    @pl.when(s == NS - 1)
    def _():
        b0 = (NS - 1) * NPT
        pltpu.sync_copy(
            acc.at[pl.ds(b0, NPT_LAST)],
            out.at[pl.ds(pl.multiple_of(c * N + b0, 8), NPT_LAST)])


def _relu_layer(agg_ref, deg_ref, h_ref, root_ref, bias_ref):
    a = agg_ref[0] + agg_ref[1]
    deg = deg_ref[0, :, 0:1] + deg_ref[1, :, 0:1]
    a = a / jnp.maximum(deg, 1.0)
    return jax.nn.relu(
        a + jnp.dot(h_ref[...], root_ref[...],
                    preferred_element_type=jnp.float32) + bias_ref[...])


def _lem_body(agg_ref, deg_ref, h_ref, root_ref, bias_ref, wr_ref,
              hn_ref, tab_ref):
    hn = _relu_layer(agg_ref, deg_ref, h_ref, root_ref, bias_ref)
    hn_ref[...] = hn
    tab_ref[...] = jnp.dot(hn, wr_ref[...],
                           preferred_element_type=jnp.float32)


_lem_call = pl.pallas_call(
    _lem_body,
    grid=(N // _NB,),
    in_specs=[pl.BlockSpec((NC, _NB, F), lambda i: (0, i, 0)),
              pl.BlockSpec((NC, _NB, F), lambda i: (0, i, 0)),
              pl.BlockSpec((_NB, F), lambda i: (i, 0)),
              pl.BlockSpec((F, F), lambda i: (0, 0)),
              pl.BlockSpec((1, F), lambda i: (0, 0)),
              pl.BlockSpec((F, K * F), lambda i: (0, 0))],
    out_specs=[pl.BlockSpec((_NB, F), lambda i: (i, 0)),
               pl.BlockSpec((_NB, K * F), lambda i: (i, 0))],
    out_shape=[jax.ShapeDtypeStruct((N, F), jnp.float32),
               jax.ShapeDtypeStruct((N, K * F), jnp.float32)],
)


def _lef_body(agg_ref, deg_ref, h_ref, root_ref, bias_ref, fw_ref, fb_ref,
              o_ref):
    hn = _relu_layer(agg_ref, deg_ref, h_ref, root_ref, bias_ref)
    o_ref[...] = jax.nn.sigmoid(
        jnp.dot(hn, fw_ref[...],
                preferred_element_type=jnp.float32) + fb_ref[...])


_lef_call = pl.pallas_call(
    _lef_body,
    grid=(N // _NB,),
    in_specs=[pl.BlockSpec((NC, _NB, F), lambda i: (0, i, 0)),
              pl.BlockSpec((NC, _NB, F), lambda i: (0, i, 0)),
              pl.BlockSpec((_NB, F), lambda i: (i, 0)),
              pl.BlockSpec((F, F), lambda i: (0, 0)),
              pl.BlockSpec((1, F), lambda i: (0, 0)),
              pl.BlockSpec((F, 1), lambda i: (0, 0)),
              pl.BlockSpec((1, 1), lambda i: (0, 0))],
    out_specs=pl.BlockSpec((_NB, 1), lambda i: (i, 0)),
    out_shape=jax.ShapeDtypeStruct((N, 1), jnp.float32),
)


def kernel(x, edge_index, edge_attr, W1, root1, b1, W2, root2, b2,
           W3, root3, b3, fc_w, fc_b):
    f32 = jnp.float32
    eat = edge_attr.T

    gflat, bflat, dstc = _basis_call(eat, edge_index)

    zeros = jnp.zeros((N, F), f32)
    ones128 = jnp.ones((128, F), f32)

    xp = jnp.pad(x, ((0, 0), (0, F - 2)))
    Wr1 = jnp.pad(jnp.transpose(W1, (1, 0, 2)).reshape(2, K * F),
                  ((0, F - 2), (0, 0)))
    r1p = jnp.pad(root1, ((0, F - 2), (0, 0)))
    Wr2 = jnp.transpose(W2, (1, 0, 2)).reshape(F, K * F)
    Wr3 = jnp.transpose(W3, (1, 0, 2)).reshape(F, K * F)

    tab = _mm_call(xp, Wr1)
    h = xp
    aggf, degf = _layer1_call(gflat, bflat, dstc, tab.reshape(N * K, F),
                              zeros, ones128)
    deg3 = degf.reshape(NC, N, F)
    h, tab = _lem_call(aggf.reshape(NC, N, F), deg3, h, r1p,
                       b1.reshape(1, F), Wr2)
    aggf = _layer_call(gflat, bflat, dstc, tab.reshape(N * K, F), zeros)
    h, tab = _lem_call(aggf.reshape(NC, N, F), deg3, h, root2,
                       b2.reshape(1, F), Wr3)

    aggf = _layer_call(gflat, bflat, dstc, tab.reshape(N * K, F), zeros)
    out = _lef_call(aggf.reshape(NC, N, F), deg3, h, root3,
                    b3.reshape(1, F), fc_w, fc_b.reshape(1, 1))
    return out[:, 0]


# speedup vs baseline: 19.5673x; 1.0009x over previous
import functools

import jax
import jax.numpy as jnp
from jax import lax
from jax.experimental import pallas as pl
from jax.experimental.pallas import tpu as pltpu
from jax.experimental.pallas import tpu_sc as plsc

N = 10000
E = 160000
K = 25
F = 16

NC, NS = 2, 16
NW = NC * NS
R = E * 4
GR = R // 128
CPW = GR // NW
GREM = GR - CPW * NW
ER = E // 128
RPC = ER // NC
RPT = RPC // NS
NPT = 624
NPT_LAST = N - (NS - 1) * NPT

_mesh = plsc.VectorSubcoreMesh(
    core_axis_name="c", subcore_axis_name="s", num_cores=NC, num_subcores=NS)


def _basis_body(eat_ref, ei_ref, g_ref, b_ref, d_ref):
    ea0 = eat_ref[0, :]
    ea1 = eat_ref[1, :]
    src = ei_ref[0, :]
    v0 = ea0 * 4.0
    v1 = ea1 * 4.0
    lo0 = jnp.floor(v0)
    lo1 = jnp.floor(v1)
    f0 = v0 - lo0
    f1 = v1 - lo1
    i0 = lo0.astype(jnp.int32)
    i1 = lo1.astype(jnp.int32)
    i0b = jnp.minimum(i0 + 1, 4)
    i1b = jnp.minimum(i1 + 1, 4)
    s25 = src * K
    b_ref[...] = jnp.concatenate(
        [((1 - f0) * (1 - f1))[None], ((1 - f0) * f1)[None],
         (f0 * (1 - f1))[None], (f0 * f1)[None]], axis=0)
    g_ref[...] = jnp.concatenate(
        [(s25 + i0 + 5 * i1)[None], (s25 + i0 + 5 * i1b)[None],
         (s25 + i0b + 5 * i1)[None], (s25 + i0b + 5 * i1b)[None]], axis=0)
    d_ref[...] = ei_ref[1:2, :]


_EB = 16000
_NEB = E // _EB
_basis_call = pl.pallas_call(
    _basis_body,
    grid=(_NEB,),
    in_specs=[pl.BlockSpec((2, _EB), lambda i: (0, i)),
              pl.BlockSpec((2, _EB), lambda i: (0, i))],
    out_specs=[pl.BlockSpec((4, _EB), lambda i: (0, i)),
               pl.BlockSpec((4, _EB), lambda i: (0, i)),
               pl.BlockSpec((1, _EB), lambda i: (0, i))],
    out_shape=[jax.ShapeDtypeStruct((4, E), jnp.int32),
               jax.ShapeDtypeStruct((4, E), jnp.float32),
               jax.ShapeDtypeStruct((1, E), jnp.int32)],
)


def _mm_body(h_ref, w_ref, o_ref):
    o_ref[...] = jnp.dot(h_ref[...], w_ref[...],
                         preferred_element_type=jnp.float32)


_NB = 2000
_mm_call = pl.pallas_call(
    _mm_body,
    grid=(N // _NB,),
    in_specs=[pl.BlockSpec((_NB, F), lambda i: (i, 0)),
              pl.BlockSpec((F, K * F), lambda i: (0, 0))],
    out_specs=pl.BlockSpec((_NB, K * F), lambda i: (i, 0)),
    out_shape=jax.ShapeDtypeStruct((N, K * F), jnp.float32),
)


SEG = (RPT + 1) * 128

_GATHER_DNUMS = lax.GatherDimensionNumbers(
    offset_dims=(), collapsed_slice_dims=(0,), start_index_map=(0,))


def _bcast_lane(vec, lane):
    return lax.gather(vec, lane[:, None], _GATHER_DNUMS, slice_sizes=(1,),
                      mode=lax.GatherScatterMode.PROMISE_IN_BOUNDS)


def _weight_chunk(gbuf, bvals, mbuf, p, b):
    for t in range(8):
        bv = [bvals[pl.ds(j * SEG + b * 128 + t * 16, 16)] for j in range(4)]

        @plsc.parallel_loop(0, 16, unroll=4)
        def _(l):
            base = p * 512 + t * 16 + l
            lane = jnp.full((16,), l, jnp.int32)
            g0 = gbuf[base, :]
            g1 = gbuf[base + 128, :]
            g2 = gbuf[base + 256, :]
            g3 = gbuf[base + 384, :]
            w0 = _bcast_lane(bv[0], lane)
            w1 = _bcast_lane(bv[1], lane)
            w2 = _bcast_lane(bv[2], lane)
            w3 = _bcast_lane(bv[3], lane)
            mbuf[p * 128 + t * 16 + l, :] = (
                (w0 * g0 + w1 * g1) + (w2 * g2 + w3 * g3))


def _layer_impl(gflat, bflat, dstc, table, zeros, out,
                idxv, bvals, didx, gbuf, mbuf, acc,
                gsem0, gsem1, ssem0, ssem1,
                ones128=None, degout=None, obuf=None, acc2=None):
    do_deg = degout is not None
    c = lax.axis_index("c")
    s = lax.axis_index("s")
    rbase = c * RPC + s * RPT
    ebase = pl.multiple_of(rbase * 128, 128)
    nb = RPT + jnp.where(s == NS - 1, 1, 0)

    cps = []
    for j in range(4):
        cps.append(pltpu.async_copy(gflat.at[j, pl.ds(ebase, RPT * 128)],
                                    idxv.at[pl.ds(j * SEG, RPT * 128)],
                                    gsem0))
        cps.append(pltpu.async_copy(bflat.at[j, pl.ds(ebase, RPT * 128)],
                                    bvals.at[pl.ds(j * SEG, RPT * 128)],
                                    gsem1))

    @pl.when(s == NS - 1)
    def _():
        e2 = pl.multiple_of((c * RPC + NS * RPT) * 128, 128)
        tcps = []
        for j in range(4):
            tcps.append(pltpu.async_copy(
                gflat.at[j, pl.ds(e2, 128)],
                idxv.at[pl.ds(j * SEG + RPT * 128, 128)], gsem0))
            tcps.append(pltpu.async_copy(
                bflat.at[j, pl.ds(e2, 128)],
                bvals.at[pl.ds(j * SEG + RPT * 128, 128)], gsem1))
        for cp in tcps:
            cp.wait()

    if do_deg:
        pltpu.sync_copy(ones128, obuf)
        _init_acc(zeros, acc2, s)
    _init_acc(zeros, acc, s)
    for cp in cps:
        cp.wait()
    plsc.subcore_barrier()

    def _fire(b, p, sem):
        for j in range(4):
            pltpu.async_copy(
                table.at[idxv.at[pl.ds(pl.multiple_of(j * SEG + b * 128, 128),
                                       128)]],
                gbuf.at[pl.ds(p * 512 + j * 128, 128)], sem)
        pltpu.async_copy(
            dstc.at[0, pl.ds(pl.multiple_of((rbase + b) * 128, 128), 128)],
            didx.at[lax.rem(b, 4)], sem)

    def _drain(b, p, sem):
        for j in range(4):
            pltpu.make_async_copy(
                table.at[idxv.at[pl.ds(pl.multiple_of(j * SEG + b * 128, 128),
                                       128)]],
                gbuf.at[pl.ds(p * 512 + j * 128, 128)], sem).wait()
        pltpu.make_async_copy(
            dstc.at[0, pl.ds(pl.multiple_of((rbase + b) * 128, 128), 128)],
            didx.at[lax.rem(b, 4)], sem).wait()

    _fire(0, 0, gsem0)

    def step(b, carry):
        p = lax.rem(b, 2)

        def _body(p_lit, gsem, ssem):
            @pl.when(b >= 2)
            def _():
                pltpu.make_async_copy(
                    mbuf.at[pl.ds(p_lit * 128, 128)],
                    acc.at[didx.at[lax.rem(b - 2, 4)]], ssem).wait()
                if do_deg:
                    pltpu.make_async_copy(
                        obuf, acc2.at[didx.at[lax.rem(b - 2, 4)]],
                        ssem).wait()

            @pl.when(b + 1 < nb)
            def _():
                _fire(b + 1, 1 - p_lit, gsem1 if p_lit == 0 else gsem0)

            _drain(b, p_lit, gsem)
            _weight_chunk(gbuf, bvals, mbuf, p_lit, b)
            pltpu.async_copy(mbuf.at[pl.ds(p_lit * 128, 128)],
                             acc.at[didx.at[lax.rem(b, 4)]], ssem,
                             add=True)
            if do_deg:
                pltpu.async_copy(obuf, acc2.at[didx.at[lax.rem(b, 4)]],
                                 ssem, add=True)

        @pl.when(p == 0)
        def _():
            _body(0, gsem0, ssem0)

        @pl.when(p == 1)
        def _():
            _body(1, gsem1, ssem1)

        return carry

    lax.fori_loop(0, nb, step, 0)
    pltpu.make_async_copy(mbuf.at[pl.ds(0, 128)],
                          acc.at[didx.at[0]], ssem0).wait()
    pltpu.make_async_copy(mbuf.at[pl.ds(128, 128)],
                          acc.at[didx.at[1]], ssem1).wait()
    if do_deg:
        pltpu.make_async_copy(obuf, acc2.at[didx.at[0]], ssem0).wait()
        pltpu.make_async_copy(obuf, acc2.at[didx.at[1]], ssem1).wait()
    plsc.subcore_barrier()
    _copy_out(acc, out, c, s)
    if do_deg:
        _copy_out(acc2, degout, c, s)


def _layer_body(gflat, bflat, dstc, table, zeros, out,
                idxv, bvals, didx, gbuf, mbuf, acc,
                gsem0, gsem1, ssem0, ssem1):
    _layer_impl(gflat, bflat, dstc, table, zeros, out,
                idxv, bvals, didx, gbuf, mbuf, acc,
                gsem0, gsem1, ssem0, ssem1)


def _layer1_body(gflat, bflat, dstc, table, zeros, ones128, out, degout,
                 idxv, bvals, didx, gbuf, mbuf, obuf, acc, acc2,
                 gsem0, gsem1, ssem0, ssem1):
    _layer_impl(gflat, bflat, dstc, table, zeros, out,
                idxv, bvals, didx, gbuf, mbuf, acc,
                gsem0, gsem1, ssem0, ssem1,
                ones128=ones128, degout=degout, obuf=obuf, acc2=acc2)


_SC_PARAMS = pltpu.CompilerParams(use_tc_tiling_on_sc=False,
                                  needs_layout_passes=False)
_SC_SCRATCH = [pltpu.VMEM((4 * SEG,), jnp.int32),
               pltpu.VMEM((4 * SEG,), jnp.float32),
               pltpu.VMEM((4, 128), jnp.int32),
               pltpu.VMEM((1024, F), jnp.float32),
               pltpu.VMEM((256, F), jnp.float32),
               pltpu.VMEM_SHARED((N, F), jnp.float32),
               pltpu.SemaphoreType.DMA,
               pltpu.SemaphoreType.DMA,
               pltpu.SemaphoreType.DMA,
               pltpu.SemaphoreType.DMA]

_layer_call = functools.partial(
    pl.kernel,
    out_type=jax.ShapeDtypeStruct((NC * N, F), jnp.float32),
    mesh=_mesh,
    compiler_params=_SC_PARAMS,
    scratch_types=_SC_SCRATCH,
)(_layer_body)

_layer1_call = functools.partial(
    pl.kernel,
    out_type=[jax.ShapeDtypeStruct((NC * N, F), jnp.float32),
              jax.ShapeDtypeStruct((NC * N, F), jnp.float32)],
    mesh=_mesh,
    compiler_params=_SC_PARAMS,
    scratch_types=[pltpu.VMEM((4 * SEG,), jnp.int32),
                   pltpu.VMEM((4 * SEG,), jnp.float32),
                   pltpu.VMEM((4, 128), jnp.int32),
                   pltpu.VMEM((1024, F), jnp.float32),
                   pltpu.VMEM((256, F), jnp.float32),
                   pltpu.VMEM((128, F), jnp.float32),
                   pltpu.VMEM_SHARED((N, F), jnp.float32),
                   pltpu.VMEM_SHARED((N, F), jnp.float32),
                   pltpu.SemaphoreType.DMA,
                   pltpu.SemaphoreType.DMA,
                   pltpu.SemaphoreType.DMA,
                   pltpu.SemaphoreType.DMA],
)(_layer1_body)


def _init_acc(zeros, acc, s):
    nb = pl.multiple_of(s * NPT, 8)

    @pl.when(s < NS - 1)
    def _():
        pltpu.sync_copy(zeros.at[pl.ds(nb, NPT)], acc.at[pl.ds(nb, NPT)])

    @pl.when(s == NS - 1)
    def _():
        b0 = (NS - 1) * NPT
        pltpu.sync_copy(zeros.at[pl.ds(b0, NPT_LAST)],
                        acc.at[pl.ds(b0, NPT_LAST)])


def _copy_out(acc, out, c, s):
    nb = pl.multiple_of(s * NPT, 8)

    @pl.when(s < NS - 1)
    def _():
        pltpu.sync_copy(acc.at[pl.ds(nb, NPT)],
                        out.at[pl.ds(pl.multiple_of(c * N + nb, 8), NPT)])

    @pl.when(s == NS - 1)
    def _():
        b0 = (NS - 1) * NPT
        pltpu.sync_copy(
            acc.at[pl.ds(b0, NPT_LAST)],
            out.at[pl.ds(pl.multiple_of(c * N + b0, 8), NPT_LAST)])


def _relu_layer(agg_ref, deg_ref, h_ref, root_ref, bias_ref):
    a = agg_ref[0] + agg_ref[1]
    deg = deg_ref[0, :, 0:1] + deg_ref[1, :, 0:1]
    a = a / jnp.maximum(deg, 1.0)
    return jax.nn.relu(
        a + jnp.dot(h_ref[...], root_ref[...],
                    preferred_element_type=jnp.float32) + bias_ref[...])


def _lem_body(agg_ref, deg_ref, h_ref, root_ref, bias_ref, wr_ref,
              hn_ref, tab_ref):
    hn = _relu_layer(agg_ref, deg_ref, h_ref, root_ref, bias_ref)
    hn_ref[...] = hn
    tab_ref[...] = jnp.dot(hn, wr_ref[...],
                           preferred_element_type=jnp.float32)


_lem_call = pl.pallas_call(
    _lem_body,
    grid=(N // _NB,),
    in_specs=[pl.BlockSpec((NC, _NB, F), lambda i: (0, i, 0)),
              pl.BlockSpec((NC, _NB, F), lambda i: (0, i, 0)),
              pl.BlockSpec((_NB, F), lambda i: (i, 0)),
              pl.BlockSpec((F, F), lambda i: (0, 0)),
              pl.BlockSpec((1, F), lambda i: (0, 0)),
              pl.BlockSpec((F, K * F), lambda i: (0, 0))],
    out_specs=[pl.BlockSpec((_NB, F), lambda i: (i, 0)),
               pl.BlockSpec((_NB, K * F), lambda i: (i, 0))],
    out_shape=[jax.ShapeDtypeStruct((N, F), jnp.float32),
               jax.ShapeDtypeStruct((N, K * F), jnp.float32)],
)


def _lef_body(agg_ref, deg_ref, h_ref, root_ref, bias_ref, fw_ref, fb_ref,
              o_ref):
    hn = _relu_layer(agg_ref, deg_ref, h_ref, root_ref, bias_ref)
    o_ref[...] = jax.nn.sigmoid(
        jnp.dot(hn, fw_ref[...],
                preferred_element_type=jnp.float32) + fb_ref[...])


_lef_call = pl.pallas_call(
    _lef_body,
    grid=(N // _NB,),
    in_specs=[pl.BlockSpec((NC, _NB, F), lambda i: (0, i, 0)),
              pl.BlockSpec((NC, _NB, F), lambda i: (0, i, 0)),
              pl.BlockSpec((_NB, F), lambda i: (i, 0)),
              pl.BlockSpec((F, F), lambda i: (0, 0)),
              pl.BlockSpec((1, F), lambda i: (0, 0)),
              pl.BlockSpec((F, 1), lambda i: (0, 0)),
              pl.BlockSpec((1, 1), lambda i: (0, 0))],
    out_specs=pl.BlockSpec((_NB, 1), lambda i: (i, 0)),
    out_shape=jax.ShapeDtypeStruct((N, 1), jnp.float32),
)


def kernel(x, edge_index, edge_attr, W1, root1, b1, W2, root2, b2,
           W3, root3, b3, fc_w, fc_b):
    f32 = jnp.float32
    eat = edge_attr.T

    gflat, bflat, dstc = _basis_call(eat, edge_index)

    zeros = jnp.zeros((N, F), f32)
    ones128 = jnp.ones((128, F), f32)

    xp = jnp.pad(x, ((0, 0), (0, F - 2)))
    Wr1 = jnp.pad(jnp.transpose(W1, (1, 0, 2)).reshape(2, K * F),
                  ((0, F - 2), (0, 0)))
    r1p = jnp.pad(root1, ((0, F - 2), (0, 0)))
    Wr2 = jnp.transpose(W2, (1, 0, 2)).reshape(F, K * F)
    Wr3 = jnp.transpose(W3, (1, 0, 2)).reshape(F, K * F)

    tab = _mm_call(xp, Wr1)
    h = xp
    aggf, degf = _layer1_call(gflat, bflat, dstc, tab.reshape(N * K, F),
                              zeros, ones128)
    deg3 = degf.reshape(NC, N, F)
    h, tab = _lem_call(aggf.reshape(NC, N, F), deg3, h, r1p,
                       b1.reshape(1, F), Wr2)
    aggf = _layer_call(gflat, bflat, dstc, tab.reshape(N * K, F), zeros)
    h, tab = _lem_call(aggf.reshape(NC, N, F), deg3, h, root2,
                       b2.reshape(1, F), Wr3)

    aggf = _layer_call(gflat, bflat, dstc, tab.reshape(N * K, F), zeros)
    out = _lef_call(aggf.reshape(NC, N, F), deg3, h, root3,
                    b3.reshape(1, F), fc_w, fc_b.reshape(1, 1))
    return out[:, 0]
